# baseline (speedup 1.0000x reference)
import functools

import jax
import jax.numpy as jnp
from jax.experimental import pallas as pl
from jax.experimental.pallas import tpu as pltpu

_LANE = 128


def _se_body(x_ref, w1_ref, b1_ref, w2_ref, b2_ref, o_ref, *, hw, inv_hw):
    x = x_ref[...]
    lane = jax.lax.broadcasted_iota(jnp.int32, x.shape, 2)
    xm = jnp.where(lane < hw, x, 0.0)
    pooled = jnp.sum(xm, axis=2) * inv_hw
    h = jnp.dot(pooled, w1_ref[...],
                preferred_element_type=jnp.float32) + b1_ref[...]
    h = h * jax.nn.sigmoid(h)
    s = jnp.dot(h, w2_ref[...],
                preferred_element_type=jnp.float32) + b2_ref[...]
    g = jax.nn.sigmoid(s)
    o_ref[...] = x[:, :, :hw] * g[:, :, None]


@jax.jit
def kernel(x, w1, b1, w2, b2):
    B, C, H, W = x.shape
    R = w1.shape[0]
    HW = H * W
    HWP = -(-HW // _LANE) * _LANE

    x3 = x.reshape(B, C, HW)
    xp = jnp.pad(x3, ((0, 0), (0, 0), (0, HWP - HW)))
    w1t = jnp.asarray(w1, jnp.float32).T
    w2t = jnp.asarray(w2, jnp.float32).T
    b1r = jnp.asarray(b1, jnp.float32).reshape(1, R)
    b2r = jnp.asarray(b2, jnp.float32).reshape(1, C)

    TB = 8
    body = functools.partial(_se_body, hw=HW, inv_hw=1.0 / HW)
    out = pl.pallas_call(
        body,
        out_shape=jax.ShapeDtypeStruct((B, C, HW), x.dtype),
        grid=(B // TB,),
        in_specs=[
            pl.BlockSpec((TB, C, HWP), lambda b: (b, 0, 0)),
            pl.BlockSpec((C, R), lambda b: (0, 0)),
            pl.BlockSpec((1, R), lambda b: (0, 0)),
            pl.BlockSpec((R, C), lambda b: (0, 0)),
            pl.BlockSpec((1, C), lambda b: (0, 0)),
        ],
        out_specs=pl.BlockSpec((TB, C, HW), lambda b: (b, 0, 0)),
        compiler_params=pltpu.CompilerParams(
            dimension_semantics=("parallel",),
            vmem_limit_bytes=60 << 20,
        ),
        cost_estimate=pl.CostEstimate(
            flops=int(B * C * HW + 4 * B * C * R),
            transcendentals=int(B * (R + C)),
            bytes_accessed=int(2 * B * C * HW * 4),
        ),
    )(xp, w1t, b1r, w2t, b2r)
    return out.reshape(B, C, H, W)
